# baseline (speedup 1.0000x reference)
import functools

import jax
import jax.numpy as jnp
from jax import lax
from jax.experimental import pallas as pl
from jax.experimental.pallas import tpu as pltpu
from jax.experimental.pallas import tpu_sc as plsc

N_NODES = 10000
PAD_N = 10240
D = 128
N_EDGES = 320000

NC = 2
NS = 16
NW = NC * NS

ROWS_PER_TILE = PAD_N // NS

CHUNK = 128
N_CHUNKS = 80
EPW = N_CHUNKS * CHUNK
E_PAD = NW * EPW
N_PAIRS = N_CHUNKS // 2 - 1

DCHUNK = 80
DEPW = N_EDGES // NW
DN_CHUNKS = DEPW // DCHUNK

_SC_MESH = plsc.VectorSubcoreMesh(core_axis_name="c", subcore_axis_name="s")


def _fill_const(ref, rows, cols, val):
  v = jnp.full((16,), val, dtype=jnp.float32)
  for i in range(rows):
    for j in range(cols // 16):
      ref[i, pl.ds(j * 16, 16)] = v


@functools.partial(
    pl.kernel, mesh=_SC_MESH,
    out_type=jax.ShapeDtypeStruct((NC, PAD_N, D), jnp.float32),
    scratch_types=[
        pltpu.VMEM((EPW,), jnp.int32),
        pltpu.VMEM((CHUNK,), jnp.int32),
        pltpu.VMEM((CHUNK,), jnp.int32),
        pltpu.VMEM((CHUNK, D), jnp.float32),
        pltpu.VMEM((CHUNK, D), jnp.float32),
        pltpu.VMEM_SHARED((PAD_N, D), jnp.float32),
        pltpu.SemaphoreType.DMA,
        pltpu.SemaphoreType.DMA,
    ])
def _seg(y_hbm, src_hbm, dst_hbm, z_hbm, agg_out, src_a, dst0, dst1,
         rows0, rows1, agg_sh, sem0, sem1):
  c = lax.axis_index("c")
  s = lax.axis_index("s")
  wid = s * NC + c
  rbase = s * ROWS_PER_TILE
  ebase = wid * EPW

  pltpu.sync_copy(z_hbm.at[pl.ds(rbase, ROWS_PER_TILE)],
                  agg_sh.at[pl.ds(rbase, ROWS_PER_TILE)])
  pltpu.sync_copy(src_hbm.at[pl.ds(ebase, EPW)], src_a)
  plsc.subcore_barrier()

  def gather(k, rows, sem):
    pltpu.async_copy(y_hbm.at[src_a.at[pl.ds(k * CHUNK, CHUNK)]], rows, sem)

  def gwait(k, rows, sem):
    pltpu.make_async_copy(y_hbm.at[src_a.at[pl.ds(k * CHUNK, CHUNK)]],
                          rows, sem).wait()

  def dstcopy(k, buf):
    pltpu.sync_copy(dst_hbm.at[pl.ds(ebase + k * CHUNK, CHUNK)], buf)

  dstcopy(0, dst0)
  gather(0, rows0, sem0)
  dstcopy(1, dst1)

  def body(p, carry):
    a = 2 * p
    gather(a + 1, rows1, sem1)
    gwait(a, rows0, sem0)
    pltpu.sync_copy(rows0, agg_sh.at[dst0], add=True)
    dstcopy(a + 2, dst0)
    gather(a + 2, rows0, sem0)
    gwait(a + 1, rows1, sem1)
    pltpu.sync_copy(rows1, agg_sh.at[dst1], add=True)
    dstcopy(a + 3, dst1)
    return carry

  lax.fori_loop(0, N_PAIRS, body, 0)

  a = 2 * N_PAIRS
  gather(a + 1, rows1, sem1)
  gwait(a, rows0, sem0)
  pltpu.sync_copy(rows0, agg_sh.at[dst0], add=True)
  gwait(a + 1, rows1, sem1)
  pltpu.sync_copy(rows1, agg_sh.at[dst1], add=True)

  plsc.subcore_barrier()
  pltpu.sync_copy(agg_sh.at[pl.ds(rbase, ROWS_PER_TILE)],
                  agg_out.at[c, pl.ds(rbase, ROWS_PER_TILE)])


@functools.partial(
    pl.kernel, mesh=_SC_MESH,
    out_type=jax.ShapeDtypeStruct((NC, PAD_N, D), jnp.float32),
    scratch_types=[
        pltpu.VMEM((DCHUNK,), jnp.int32),
        pltpu.VMEM((DCHUNK, D), jnp.float32),
        pltpu.VMEM_SHARED((PAD_N, D), jnp.float32),
    ])
def _deg(dst_hbm, z_hbm, deg_out, dst_v, ones_v, deg_sh):
  c = lax.axis_index("c")
  s = lax.axis_index("s")
  wid = s * NC + c
  rbase = s * ROWS_PER_TILE

  _fill_const(ones_v, DCHUNK, D, 1.0)
  pltpu.sync_copy(z_hbm.at[pl.ds(rbase, ROWS_PER_TILE)],
                  deg_sh.at[pl.ds(rbase, ROWS_PER_TILE)])
  plsc.subcore_barrier()

  def body(k, carry):
    base = wid * DEPW + k * DCHUNK
    pltpu.sync_copy(dst_hbm.at[pl.ds(base, DCHUNK)], dst_v)
    pltpu.sync_copy(ones_v, deg_sh.at[dst_v], add=True)
    return carry

  lax.fori_loop(0, DN_CHUNKS, body, 0)
  plsc.subcore_barrier()

  pltpu.sync_copy(deg_sh.at[pl.ds(rbase, ROWS_PER_TILE)],
                  deg_out.at[c, pl.ds(rbase, ROWS_PER_TILE)])


_BLK = 1000


def _mm_body(x_ref, wl_ref, wr_ref, y_ref, r_ref):
  xb = x_ref[...]
  y_ref[...] = jnp.dot(xb, wl_ref[...], preferred_element_type=jnp.float32)
  r_ref[...] = jnp.dot(xb, wr_ref[...], preferred_element_type=jnp.float32)


def _mm(x, wl, wr):
  n = x.shape[0]
  grid = n // _BLK
  return pl.pallas_call(
      _mm_body,
      grid=(grid,),
      in_specs=[
          pl.BlockSpec((_BLK, D), lambda i: (i, 0)),
          pl.BlockSpec((D, D), lambda i: (0, 0)),
          pl.BlockSpec((D, D), lambda i: (0, 0)),
      ],
      out_specs=[
          pl.BlockSpec((_BLK, D), lambda i: (i, 0)),
          pl.BlockSpec((_BLK, D), lambda i: (i, 0)),
      ],
      out_shape=[
          jax.ShapeDtypeStruct((n, D), jnp.float32),
          jax.ShapeDtypeStruct((n, D), jnp.float32),
      ],
  )(x, wl, wr)


def _inv_deg(degp_ref):
  deg = degp_ref[0] + degp_ref[1]
  return 1.0 / jnp.maximum(deg, 1.0)


def _combine_mm_body(aggp, degp, r1, b1, wl, wr, y2_ref, r2_ref):
  inv = _inv_deg(degp)
  h = jnp.maximum((aggp[0] + aggp[1]) * inv + b1[...] + r1[...], 0.0)
  y2_ref[...] = jnp.dot(h, wl[...], preferred_element_type=jnp.float32)
  r2_ref[...] = jnp.dot(h, wr[...], preferred_element_type=jnp.float32)


def _combine_mm(aggp, degp, r1, b1, wl, wr):
  grid = N_NODES // _BLK
  return pl.pallas_call(
      _combine_mm_body,
      grid=(grid,),
      in_specs=[
          pl.BlockSpec((NC, _BLK, D), lambda i: (0, i, 0)),
          pl.BlockSpec((NC, _BLK, D), lambda i: (0, i, 0)),
          pl.BlockSpec((_BLK, D), lambda i: (i, 0)),
          pl.BlockSpec((1, D), lambda i: (0, 0)),
          pl.BlockSpec((D, D), lambda i: (0, 0)),
          pl.BlockSpec((D, D), lambda i: (0, 0)),
      ],
      out_specs=[
          pl.BlockSpec((_BLK, D), lambda i: (i, 0)),
          pl.BlockSpec((_BLK, D), lambda i: (i, 0)),
      ],
      out_shape=[
          jax.ShapeDtypeStruct((N_NODES, D), jnp.float32),
          jax.ShapeDtypeStruct((N_NODES, D), jnp.float32),
      ],
  )(aggp, degp, r1, b1, wl, wr)


def _final_body(aggp, degp, r2, b2, out_ref):
  inv = _inv_deg(degp)
  out_ref[...] = (aggp[0] + aggp[1]) * inv + b2[...] + r2[...]


def _final(aggp, degp, r2, b2):
  grid = N_NODES // _BLK
  return pl.pallas_call(
      _final_body,
      grid=(grid,),
      in_specs=[
          pl.BlockSpec((NC, _BLK, D), lambda i: (0, i, 0)),
          pl.BlockSpec((NC, _BLK, D), lambda i: (0, i, 0)),
          pl.BlockSpec((_BLK, D), lambda i: (i, 0)),
          pl.BlockSpec((1, D), lambda i: (0, 0)),
      ],
      out_specs=pl.BlockSpec((_BLK, D), lambda i: (i, 0)),
      out_shape=jax.ShapeDtypeStruct((N_NODES, D), jnp.float32),
  )(aggp, degp, r2, b2)


@jax.jit
def kernel(x, edge_index, W1_l, b1_l, W1_r, W2_l, b2_l, W2_r):
  ei = edge_index.astype(jnp.int32)
  src = ei[0]
  dst = ei[1]
  b1 = b1_l.reshape(1, D)
  b2 = b2_l.reshape(1, D)
  zeros = jnp.zeros((PAD_N, D), jnp.float32)

  npad = E_PAD - N_EDGES
  pad_ar = jnp.arange(npad, dtype=jnp.int32)
  src_p = jnp.concatenate([src, pad_ar % N_NODES])
  dst_p = jnp.concatenate([dst, N_NODES + pad_ar % (PAD_N - N_NODES)])

  degp = _deg(dst, zeros)
  y1, r1 = _mm(x, W1_l, W1_r)
  agg1 = _seg(y1, src_p, dst_p, zeros)
  y2, r2 = _combine_mm(agg1, degp, r1, b1, W2_l, W2_r)
  agg2 = _seg(y2, src_p, dst_p, zeros)
  out = _final(agg2, degp, r2, b2)
  return out
